# baseline (speedup 1.0000x reference)
import functools

import jax
import jax.numpy as jnp
from jax import lax
from jax.experimental import pallas as pl
from jax.experimental.pallas import tpu as pltpu
from jax.experimental.pallas import tpu_sc as plsc

B, H, W = 64, 512, 512
PIX = H * W
NC, NS = 2, 16
NW = NC * NS
BPW = B // NW
CHUNK = 16384
NCHUNK = PIX // CHUNK
LANES = 16

_mesh = plsc.VectorSubcoreMesh(core_axis_name="c", subcore_axis_name="s")


@functools.partial(
    pl.kernel,
    out_type=jax.ShapeDtypeStruct((B, 2, PIX), jnp.float32),
    mesh=_mesh,
    scratch_types=[
        pltpu.VMEM((CHUNK,), jnp.float32),
        pltpu.VMEM((CHUNK,), jnp.float32),
        pltpu.VMEM((CHUNK,), jnp.float32),
    ],
)
def _sc_mask(mask_hbm, out_hbm, in_v, bw_v, bg_v):
    wid = lax.axis_index("s") * NC + lax.axis_index("c")
    one = jnp.full((LANES,), 1.0, jnp.float32)
    zero = jnp.zeros((LANES,), jnp.float32)

    def chunk_body(t, _):
        bi = t // NCHUNK
        j = t % NCHUNK
        b = wid * BPW + bi
        pltpu.sync_copy(mask_hbm.at[b, pl.ds(j * CHUNK, CHUNK)], in_v)

        def vec_body(i, _):
            x = in_v[pl.ds(i * LANES, LANES)]
            bw_v[pl.ds(i * LANES, LANES)] = jnp.where(x > 0.0, one, zero)
            is_bg = (x == 0.0) | (x == 0.25)
            bg_v[pl.ds(i * LANES, LANES)] = jnp.where(is_bg, one, zero)
            return 0

        lax.fori_loop(0, CHUNK // LANES, vec_body, 0)
        pltpu.sync_copy(bw_v, out_hbm.at[b, 0, pl.ds(j * CHUNK, CHUNK)])
        pltpu.sync_copy(bg_v, out_hbm.at[b, 1, pl.ds(j * CHUNK, CHUNK)])
        return 0

    lax.fori_loop(0, BPW * NCHUNK, chunk_body, 0)


def kernel(mask):
    out = _sc_mask(mask.reshape(B, PIX))
    return out.reshape(B, 2, H, W)


# speedup vs baseline: 1.2534x; 1.2534x over previous
import functools

import jax
import jax.numpy as jnp
from jax import lax
from jax.experimental import pallas as pl
from jax.experimental.pallas import tpu as pltpu
from jax.experimental.pallas import tpu_sc as plsc

B, H, W = 64, 512, 512
PIX = H * W
NC, NS = 2, 16
NW = NC * NS
BPW = B // NW
CHUNK = 16384
NCHUNK = PIX // CHUNK
NT = BPW * NCHUNK
LANES = 16
UNROLL = 8

_mesh = plsc.VectorSubcoreMesh(core_axis_name="c", subcore_axis_name="s")


@functools.partial(
    pl.kernel,
    out_type=jax.ShapeDtypeStruct((B, 2, PIX), jnp.float32),
    mesh=_mesh,
    scratch_types=[
        pltpu.VMEM((2, CHUNK), jnp.float32),
        pltpu.VMEM((2, CHUNK), jnp.float32),
        pltpu.VMEM((2, CHUNK), jnp.float32),
        pltpu.SemaphoreType.DMA((2,)),
        pltpu.SemaphoreType.DMA((2,)),
        pltpu.SemaphoreType.DMA((2,)),
    ],
)
def _sc_mask(mask_hbm, out_hbm, in_v, bw_v, bg_v, in_sem, bw_sem, bg_sem):
    wid = lax.axis_index("s") * NC + lax.axis_index("c")
    base = wid * BPW
    one = jnp.full((LANES,), 1.0, jnp.float32)
    zero = jnp.zeros((LANES,), jnp.float32)

    def in_cp(t, slot):
        b = base + t // NCHUNK
        j = t % NCHUNK
        return pltpu.make_async_copy(
            mask_hbm.at[b, pl.ds(j * CHUNK, CHUNK)], in_v.at[slot],
            in_sem.at[slot])

    def out_cp(t, slot, chan, buf, sem):
        b = base + t // NCHUNK
        j = t % NCHUNK
        return pltpu.make_async_copy(
            buf.at[slot], out_hbm.at[b, chan, pl.ds(j * CHUNK, CHUNK)],
            sem.at[slot])

    def step(t, slot):
        @pl.when(t + 1 < NT)
        def _():
            in_cp(t + 1, 1 - slot).start()

        in_cp(t, slot).wait()

        @pl.when(t >= 2)
        def _():
            out_cp(t - 2, slot, 0, bw_v, bw_sem).wait()
            out_cp(t - 2, slot, 1, bg_v, bg_sem).wait()

        src = in_v.at[slot]
        dst_bw = bw_v.at[slot]
        dst_bg = bg_v.at[slot]

        @plsc.parallel_loop(0, CHUNK // LANES, unroll=UNROLL)
        def _(i):
            x = src[pl.ds(i * LANES, LANES)]
            dst_bw[pl.ds(i * LANES, LANES)] = jnp.where(x > 0.0, one, zero)
            is_bg = (x == 0.0) | (x == 0.25)
            dst_bg[pl.ds(i * LANES, LANES)] = jnp.where(is_bg, one, zero)

        out_cp(t, slot, 0, bw_v, bw_sem).start()
        out_cp(t, slot, 1, bg_v, bg_sem).start()

    in_cp(0, 0).start()

    def g_body(g, _):
        step(2 * g, 0)
        step(2 * g + 1, 1)
        return 0

    lax.fori_loop(0, NT // 2, g_body, 0)

    out_cp(NT - 2, 0, 0, bw_v, bw_sem).wait()
    out_cp(NT - 2, 0, 1, bg_v, bg_sem).wait()
    out_cp(NT - 1, 1, 0, bw_v, bw_sem).wait()
    out_cp(NT - 1, 1, 1, bg_v, bg_sem).wait()


def kernel(mask):
    out = _sc_mask(mask.reshape(B, PIX))
    return out.reshape(B, 2, H, W)


# speedup vs baseline: 6.2480x; 4.9847x over previous
import functools

import jax
import jax.numpy as jnp
from jax import lax
from jax.experimental import pallas as pl
from jax.experimental.pallas import tpu as pltpu
from jax.experimental.pallas import tpu_sc as plsc

B, H, W = 64, 512, 512
NC, NS = 2, 16
NW = NC * NS
BPW = B // NW
ROWS = 32
CHUNK = ROWS * W
NCHUNK = H // ROWS
NT = BPW * NCHUNK
LANES = 16
CGRP = W // LANES
UNROLL = 8

_mesh = plsc.VectorSubcoreMesh(core_axis_name="c", subcore_axis_name="s")


@functools.partial(
    pl.kernel,
    out_type=jax.ShapeDtypeStruct((B, 2, H, W), jnp.float32),
    mesh=_mesh,
    compiler_params=pltpu.CompilerParams(use_tc_tiling_on_sc=True),
    scratch_types=[
        pltpu.VMEM((2, ROWS, W), jnp.float32),
        pltpu.VMEM((2, ROWS, W), jnp.float32),
        pltpu.VMEM((2, ROWS, W), jnp.float32),
        pltpu.SemaphoreType.DMA((2,)),
        pltpu.SemaphoreType.DMA((2,)),
        pltpu.SemaphoreType.DMA((2,)),
    ],
)
def _sc_mask(mask_hbm, out_hbm, in_v, bw_v, bg_v, in_sem, bw_sem, bg_sem):
    wid = lax.axis_index("s") * NC + lax.axis_index("c")
    base = wid * BPW
    one = jnp.full((LANES,), 1.0, jnp.float32)
    zero = jnp.zeros((LANES,), jnp.float32)

    def in_cp(t, slot):
        b = base + t // NCHUNK
        r0 = (t % NCHUNK) * ROWS
        return pltpu.make_async_copy(
            mask_hbm.at[b, pl.ds(r0, ROWS), :], in_v.at[slot],
            in_sem.at[slot])

    def out_cp(t, slot, chan, buf, sem):
        b = base + t // NCHUNK
        r0 = (t % NCHUNK) * ROWS
        return pltpu.make_async_copy(
            buf.at[slot], out_hbm.at[b, chan, pl.ds(r0, ROWS), :],
            sem.at[slot])

    def step(t, slot):
        @pl.when(t + 1 < NT)
        def _():
            in_cp(t + 1, 1 - slot).start()

        in_cp(t, slot).wait()

        @pl.when(t >= 2)
        def _():
            out_cp(t - 2, slot, 0, bw_v, bw_sem).wait()
            out_cp(t - 2, slot, 1, bg_v, bg_sem).wait()

        src = in_v.at[slot]
        dst_bw = bw_v.at[slot]
        dst_bg = bg_v.at[slot]

        def row_body(r, _):
            @plsc.parallel_loop(0, CGRP, unroll=UNROLL)
            def _(c):
                x = src[r, pl.ds(c * LANES, LANES)]
                dst_bw[r, pl.ds(c * LANES, LANES)] = jnp.where(
                    x > 0.0, one, zero)
                is_bg = (x == 0.0) | (x == 0.25)
                dst_bg[r, pl.ds(c * LANES, LANES)] = jnp.where(
                    is_bg, one, zero)

            return 0

        lax.fori_loop(0, ROWS, row_body, 0)

        out_cp(t, slot, 0, bw_v, bw_sem).start()
        out_cp(t, slot, 1, bg_v, bg_sem).start()

    in_cp(0, 0).start()

    def g_body(g, _):
        step(2 * g, 0)
        step(2 * g + 1, 1)
        return 0

    lax.fori_loop(0, NT // 2, g_body, 0)

    out_cp(NT - 2, 0, 0, bw_v, bw_sem).wait()
    out_cp(NT - 2, 0, 1, bg_v, bg_sem).wait()
    out_cp(NT - 1, 1, 0, bw_v, bw_sem).wait()
    out_cp(NT - 1, 1, 1, bg_v, bg_sem).wait()


def kernel(mask):
    return _sc_mask(mask.reshape(B, H, W))
